# Initial kernel scaffold; baseline (speedup 1.0000x reference)
import jax
import jax.numpy as jnp
from jax.experimental import pallas as pl


def kernel(inputs, embedding_weights, field_weights, linear_weights, bias_weight):
    raise NotImplementedError("write your pallas kernel here")


# speedup vs baseline: 13.7233x; 13.7233x over previous
import functools

import jax
import jax.numpy as jnp
import numpy as np
from jax import lax
from jax.experimental import pallas as pl
from jax.experimental.pallas import tpu as pltpu
from jax.experimental.pallas import tpu_sc as plsc

B = 4096
F = 26
D = 16
FD = F * D

NC = 2
NS = 16
NW = NC * NS
SAMP_PER_W = B // NW
IDX_PER_W = SAMP_PER_W * F
CHUNK = 128
NCH = IDX_PER_W // CHUNK

@functools.cache
def _get_sc_gather():
    mesh = plsc.VectorSubcoreMesh(core_axis_name="c", subcore_axis_name="s")

    @functools.partial(
        pl.kernel,
        mesh=mesh,
        compiler_params=pltpu.CompilerParams(use_tc_tiling_on_sc=False),
        out_type=[
            jax.ShapeDtypeStruct((NW, NCH, CHUNK, D), jnp.float32),
            jax.ShapeDtypeStruct((NW, NCH, CHUNK), jnp.float32),
        ],
        scratch_types=[
            pltpu.VMEM((NCH, CHUNK), jnp.int32),
            pltpu.VMEM((NCH, CHUNK, D), jnp.float32),
            pltpu.VMEM((NCH, CHUNK), jnp.float32),
            pltpu.SemaphoreType.DMA,
            pltpu.SemaphoreType.DMA,
        ],
    )
    def _sc_gather(idx_hbm, emb_hbm, lw_hbm, out_emb, out_lw,
                   idx_v, rows_v, lwv_v, sem_e, sem_l):
        wid = lax.axis_index("s") * NC + lax.axis_index("c")
        pltpu.sync_copy(idx_hbm.at[wid], idx_v)
        descs = []
        for j in range(NCH):
            descs.append(pltpu.async_copy(emb_hbm.at[idx_v.at[j]], rows_v.at[j], sem_e))
            descs.append(pltpu.async_copy(lw_hbm.at[idx_v.at[j]], lwv_v.at[j], sem_l))
        for de in descs:
            de.wait()
        pltpu.sync_copy(rows_v, out_emb.at[wid])
        pltpu.sync_copy(lwv_v, out_lw.at[wid])

    return _sc_gather


BT = 512


def _tc_body(x_ref, m_ref, lw_ref, bias_ref, o_ref):
    x = x_ref[...]
    t = jnp.dot(x, m_ref[...], preferred_element_type=jnp.float32)
    second = jnp.sum(x * t, axis=1, keepdims=True)
    first = jnp.sum(lw_ref[...], axis=1, keepdims=True)
    o_ref[...] = first + second + bias_ref[0, 0]


_I, _J = np.triu_indices(F, 1)


def kernel(inputs, embedding_weights, field_weights, linear_weights, bias_weight):
    idx2 = inputs.reshape(NW, NCH, CHUNK)
    gathered, lw_g = _get_sc_gather()(idx2, embedding_weights, linear_weights)
    xg = gathered.reshape(B, FD)
    lwg = lw_g.reshape(B, F)

    w = jnp.zeros((F, F), jnp.float32).at[_I, _J].set(field_weights[:, 0])
    m = jnp.kron(w + w.T, 0.5 * jnp.eye(D, dtype=jnp.float32))

    out = pl.pallas_call(
        _tc_body,
        grid=(B // BT,),
        in_specs=[
            pl.BlockSpec((BT, FD), lambda i: (i, 0)),
            pl.BlockSpec((FD, FD), lambda i: (0, 0)),
            pl.BlockSpec((BT, F), lambda i: (i, 0)),
            pl.BlockSpec((1, 1), lambda i: (0, 0)),
        ],
        out_specs=pl.BlockSpec((BT, 1), lambda i: (i, 0)),
        out_shape=jax.ShapeDtypeStruct((B, 1), jnp.float32),
    )(xg, m, lwg, bias_weight.reshape(1, 1))
    return out
